# baseline (speedup 1.0000x reference)
import functools

import jax
import jax.numpy as jnp
from jax import lax
from jax.experimental import pallas as pl
from jax.experimental.pallas import tpu as pltpu
from jax.experimental.pallas import tpu_sc as plsc

N_PAD = 10240
D = 128
NC, NS = 2, 16
NW = NC * NS
ROWS_PER_TILE = N_PAD // NS
SLABS = 84
SLAB_W = 128
LANE = 128
DUMMY = 10200
ROW_BLK = 2048


def _mesh():
    return plsc.VectorSubcoreMesh(core_axis_name="c", subcore_axis_name="s")


def _deg_body(dst_hbm, deg_out, idx_v, ones_v, zeros_v, deg_sh, sem):
    c = lax.axis_index("c")
    s = lax.axis_index("s")
    w = c * NS + s
    for i in range(SLAB_W // 16):
        ones_v[pl.ds(i * 16, 16)] = jnp.full((16,), 1.0, jnp.float32)
    for i in range(ROWS_PER_TILE // 16):
        zeros_v[pl.ds(i * 16, 16)] = jnp.zeros((16,), jnp.float32)
    pltpu.sync_copy(zeros_v, deg_sh.at[pl.ds(s * ROWS_PER_TILE, ROWS_PER_TILE)])
    pltpu.async_copy(dst_hbm.at[w], idx_v, sem).wait()
    plsc.subcore_barrier()

    @pl.loop(0, SLABS)
    def _(j):
        pltpu.sync_copy(ones_v, deg_sh.at[idx_v.at[j]], add=True)

    plsc.subcore_barrier()
    pltpu.sync_copy(deg_sh.at[pl.ds(s * ROWS_PER_TILE, ROWS_PER_TILE)],
                    deg_out.at[c, pl.ds(s * ROWS_PER_TILE, ROWS_PER_TILE)])


@functools.partial(jax.jit)
def _deg_call(dst_slab):
    f = pl.kernel(
        _deg_body,
        out_type=jax.ShapeDtypeStruct((NC, N_PAD), jnp.float32),
        mesh=_mesh(),
        scratch_types=[
            pltpu.VMEM((SLABS, SLAB_W), jnp.int32),
            pltpu.VMEM((SLAB_W,), jnp.float32),
            pltpu.VMEM((ROWS_PER_TILE,), jnp.float32),
            pltpu.VMEM_SHARED((N_PAD,), jnp.float32),
            pltpu.SemaphoreType.DMA,
        ],
    )
    return f(dst_slab)


def _y_body(x_ref, w_ref, deg_ref, y_ref, dinv_ref):
    d = deg_ref[0] + deg_ref[1]
    dinv = jnp.where(d > 0.0, lax.rsqrt(jnp.maximum(d, 1.0)), 0.0)
    xw = jnp.dot(x_ref[...], w_ref[...], preferred_element_type=jnp.float32)
    y_ref[...] = xw * dinv
    dinv_ref[...] = dinv


@jax.jit
def _y_call(x_pad, W, deg3):
    grid = N_PAD // ROW_BLK
    return pl.pallas_call(
        _y_body,
        grid=(grid,),
        in_specs=[
            pl.BlockSpec((ROW_BLK, D), lambda i: (i, 0)),
            pl.BlockSpec((D, D), lambda i: (0, 0)),
            pl.BlockSpec((NC, ROW_BLK, 1), lambda i: (0, i, 0)),
        ],
        out_specs=[
            pl.BlockSpec((ROW_BLK, D), lambda i: (i, 0)),
            pl.BlockSpec((ROW_BLK, 1), lambda i: (i, 0)),
        ],
        out_shape=[
            jax.ShapeDtypeStruct((N_PAD, D), jnp.float32),
            jax.ShapeDtypeStruct((N_PAD, 1), jnp.float32),
        ],
    )(x_pad, W, deg3)


def _msg_body(y_hbm, src_hbm, dst_hbm, acc_out,
              src_v, dst_v, buf0, acc_sh, sem, sem0):
    c = lax.axis_index("c")
    s = lax.axis_index("s")
    w = c * NS + s

    @pl.loop(0, SLAB_W)
    def _(i):
        for k in range(D // 16):
            buf0[i, pl.ds(k * 16, 16)] = jnp.zeros((16,), jnp.float32)

    for t in range(ROWS_PER_TILE // SLAB_W):
        pltpu.sync_copy(buf0, acc_sh.at[pl.ds(s * ROWS_PER_TILE + t * SLAB_W, SLAB_W)])

    pltpu.async_copy(src_hbm.at[w], src_v, sem).wait()
    pltpu.async_copy(dst_hbm.at[w], dst_v, sem).wait()
    plsc.subcore_barrier()

    @pl.loop(0, SLABS)
    def _(j):
        pltpu.async_copy(y_hbm.at[src_v.at[j]], buf0, sem0).wait()
        pltpu.sync_copy(buf0, acc_sh.at[dst_v.at[j]], add=True)

    plsc.subcore_barrier()
    pltpu.sync_copy(acc_sh.at[pl.ds(s * ROWS_PER_TILE, ROWS_PER_TILE)],
                    acc_out.at[c, pl.ds(s * ROWS_PER_TILE, ROWS_PER_TILE)])


@jax.jit
def _msg_call(y, src_slab, dst_slab):
    f = pl.kernel(
        _msg_body,
        out_type=jax.ShapeDtypeStruct((NC, N_PAD, D), jnp.float32),
        mesh=_mesh(),
        scratch_types=[
            pltpu.VMEM((SLABS, SLAB_W), jnp.int32),
            pltpu.VMEM((SLABS, SLAB_W), jnp.int32),
            pltpu.VMEM((SLAB_W, D), jnp.float32),
            pltpu.VMEM_SHARED((N_PAD, D), jnp.float32),
            pltpu.SemaphoreType.DMA,
            pltpu.SemaphoreType.DMA,
        ],
    )
    return f(y, src_slab, dst_slab)


def _out_body(acc_ref, dinv_ref, o_ref):
    o_ref[...] = jnp.maximum((acc_ref[0] + acc_ref[1]) * dinv_ref[...], 0.0)


@jax.jit
def _out_call(acc, dinv):
    grid = N_PAD // ROW_BLK
    return pl.pallas_call(
        _out_body,
        grid=(grid,),
        in_specs=[
            pl.BlockSpec((NC, ROW_BLK, D), lambda i: (0, i, 0)),
            pl.BlockSpec((ROW_BLK, 1), lambda i: (i, 0)),
        ],
        out_specs=pl.BlockSpec((ROW_BLK, D), lambda i: (i, 0)),
        out_shape=jax.ShapeDtypeStruct((N_PAD, D), jnp.float32),
    )(acc, dinv)


def kernel(x, edge_index, W):
    N = x.shape[0]
    src = edge_index[0].astype(jnp.int32)
    dst = edge_index[1].astype(jnp.int32)
    loop_idx = jnp.arange(N, dtype=jnp.int32)
    src_all = jnp.concatenate([src, loop_idx])
    dst_all = jnp.concatenate([dst, loop_idx])
    n_slots = NW * SLABS * SLAB_W
    pad_n = n_slots - src_all.shape[0]
    src_all = jnp.concatenate([src_all, jnp.zeros((pad_n,), jnp.int32)])
    pad_dst = N + jnp.arange(pad_n, dtype=jnp.int32) % (N_PAD - N)
    dst_all = jnp.concatenate([dst_all, pad_dst])
    src_slab = src_all.reshape(NW, SLABS, SLAB_W)
    dst_slab = dst_all.reshape(NW, SLABS, SLAB_W)
    x_pad = jnp.pad(x, ((0, N_PAD - N), (0, 0)))

    deg_part = _deg_call(dst_slab)
    deg3 = deg_part.reshape(NC, N_PAD, 1)
    y, dinv = _y_call(x_pad, W, deg3)
    acc = _msg_call(y, src_slab, dst_slab)
    out = _out_call(acc, dinv)
    return out[:N]


# speedup vs baseline: 3.9155x; 3.9155x over previous
import functools

import jax
import jax.numpy as jnp
from jax import lax
from jax.experimental import pallas as pl
from jax.experimental.pallas import tpu as pltpu
from jax.experimental.pallas import tpu_sc as plsc

N_PAD = 10240
D = 128
NC, NS = 2, 16
NW = NC * NS
ROWS_PER_TILE = N_PAD // NS
SLABS = 81
SLAB_W = 128
LANE = 128
DUMMY = 10200
ROW_BLK = 2048


def _mesh():
    return plsc.VectorSubcoreMesh(core_axis_name="c", subcore_axis_name="s")


def _deg_body(dst_hbm, deg_out, idx_v, ones_v, zeros_v, deg_sh, sem):
    c = lax.axis_index("c")
    s = lax.axis_index("s")
    w = c * NS + s
    for i in range(SLAB_W // 16):
        ones_v[pl.ds(i * 16, 16)] = jnp.full((16,), 1.0, jnp.float32)
    for i in range(ROWS_PER_TILE // 16):
        zeros_v[pl.ds(i * 16, 16)] = jnp.zeros((16,), jnp.float32)
    pltpu.sync_copy(zeros_v, deg_sh.at[pl.ds(s * ROWS_PER_TILE, ROWS_PER_TILE)])
    pltpu.async_copy(dst_hbm.at[w], idx_v, sem).wait()
    plsc.subcore_barrier()

    @pl.loop(0, SLABS)
    def _(j):
        pltpu.sync_copy(ones_v, deg_sh.at[idx_v.at[j]], add=True)

    plsc.subcore_barrier()
    pltpu.sync_copy(deg_sh.at[pl.ds(s * ROWS_PER_TILE, ROWS_PER_TILE)],
                    deg_out.at[c, pl.ds(s * ROWS_PER_TILE, ROWS_PER_TILE)])


@functools.partial(jax.jit)
def _deg_call(dst_slab):
    f = pl.kernel(
        _deg_body,
        out_type=jax.ShapeDtypeStruct((NC, N_PAD), jnp.float32),
        mesh=_mesh(),
        scratch_types=[
            pltpu.VMEM((SLABS, SLAB_W), jnp.int32),
            pltpu.VMEM((SLAB_W,), jnp.float32),
            pltpu.VMEM((ROWS_PER_TILE,), jnp.float32),
            pltpu.VMEM_SHARED((N_PAD,), jnp.float32),
            pltpu.SemaphoreType.DMA,
        ],
    )
    return f(dst_slab)


def _y_body(x_ref, w_ref, deg_ref, y_ref, dinv_ref):
    d = deg_ref[0] + deg_ref[1]
    dinv = jnp.where(d > 0.0, lax.rsqrt(jnp.maximum(d, 1.0)), 0.0)
    xw = jnp.dot(x_ref[...], w_ref[...], preferred_element_type=jnp.float32)
    y_ref[...] = xw * dinv
    dinv_ref[...] = dinv


@jax.jit
def _y_call(x_pad, W, deg3):
    grid = N_PAD // ROW_BLK
    return pl.pallas_call(
        _y_body,
        grid=(grid,),
        in_specs=[
            pl.BlockSpec((ROW_BLK, D), lambda i: (i, 0)),
            pl.BlockSpec((D, D), lambda i: (0, 0)),
            pl.BlockSpec((NC, ROW_BLK, 1), lambda i: (0, i, 0)),
        ],
        out_specs=[
            pl.BlockSpec((ROW_BLK, D), lambda i: (i, 0)),
            pl.BlockSpec((ROW_BLK, 1), lambda i: (i, 0)),
        ],
        out_shape=[
            jax.ShapeDtypeStruct((N_PAD, D), jnp.float32),
            jax.ShapeDtypeStruct((N_PAD, 1), jnp.float32),
        ],
    )(x_pad, W, deg3)


def _msg_body(y_hbm, src_hbm, dst_hbm, acc_out,
              src_v, dst_v, buf0, acc_sh, sem, sem0):
    c = lax.axis_index("c")
    s = lax.axis_index("s")
    w = c * NS + s

    @pl.loop(0, SLAB_W)
    def _(i):
        for k in range(D // 16):
            buf0[i, pl.ds(k * 16, 16)] = jnp.zeros((16,), jnp.float32)

    for t in range(ROWS_PER_TILE // SLAB_W):
        pltpu.sync_copy(buf0, acc_sh.at[pl.ds(s * ROWS_PER_TILE + t * SLAB_W, SLAB_W)])

    pltpu.async_copy(src_hbm.at[w], src_v, sem).wait()
    pltpu.async_copy(dst_hbm.at[w], dst_v, sem).wait()
    plsc.subcore_barrier()

    @pl.loop(0, SLABS)
    def _(j):
        pltpu.async_copy(y_hbm.at[src_v.at[j]], buf0, sem0).wait()
        pltpu.sync_copy(buf0, acc_sh.at[dst_v.at[j]], add=True)

    plsc.subcore_barrier()
    pltpu.sync_copy(acc_sh.at[pl.ds(s * ROWS_PER_TILE, ROWS_PER_TILE)],
                    acc_out.at[c, pl.ds(s * ROWS_PER_TILE, ROWS_PER_TILE)])


@jax.jit
def _msg_call(y, src_slab, dst_slab):
    f = pl.kernel(
        _msg_body,
        out_type=jax.ShapeDtypeStruct((NC, N_PAD, D), jnp.float32),
        mesh=_mesh(),
        scratch_types=[
            pltpu.VMEM((SLABS, SLAB_W), jnp.int32),
            pltpu.VMEM((SLABS, SLAB_W), jnp.int32),
            pltpu.VMEM((SLAB_W, D), jnp.float32),
            pltpu.VMEM_SHARED((N_PAD, D), jnp.float32),
            pltpu.SemaphoreType.DMA,
            pltpu.SemaphoreType.DMA,
        ],
    )
    return f(y, src_slab, dst_slab)


def _out_body(acc_ref, dinv_ref, o_ref):
    o_ref[...] = jnp.maximum((acc_ref[0] + acc_ref[1]) * dinv_ref[...], 0.0)


@jax.jit
def _out_call(acc, dinv):
    grid = N_PAD // ROW_BLK
    return pl.pallas_call(
        _out_body,
        grid=(grid,),
        in_specs=[
            pl.BlockSpec((NC, ROW_BLK, D), lambda i: (0, i, 0)),
            pl.BlockSpec((ROW_BLK, 1), lambda i: (i, 0)),
        ],
        out_specs=pl.BlockSpec((ROW_BLK, D), lambda i: (i, 0)),
        out_shape=jax.ShapeDtypeStruct((N_PAD, D), jnp.float32),
    )(acc, dinv)


def kernel(x, edge_index, W):
    N = x.shape[0]
    src = edge_index[0].astype(jnp.int32)
    dst = edge_index[1].astype(jnp.int32)
    loop_idx = jnp.arange(N, dtype=jnp.int32)
    src_all = jnp.concatenate([src, loop_idx])
    dst_all = jnp.concatenate([dst, loop_idx])
    n_slots = NW * SLABS * SLAB_W
    pad_n = n_slots - src_all.shape[0]
    pad_ar = jnp.arange(pad_n, dtype=jnp.int32)
    src_all = jnp.concatenate([src_all, N + pad_ar % (N_PAD - N)])
    dst_all = jnp.concatenate([dst_all, N + (pad_n - 1 - pad_ar) % (N_PAD - N)])
    src_slab = src_all.reshape(NW, SLABS, SLAB_W)
    dst_slab = dst_all.reshape(NW, SLABS, SLAB_W)
    x_pad = jnp.pad(x, ((0, N_PAD - N), (0, 0)))

    deg_part = _deg_call(dst_slab)
    deg3 = deg_part.reshape(NC, N_PAD, 1)
    y, dinv = _y_call(x_pad, W, deg3)
    acc = _msg_call(y, src_slab, dst_slab)
    out = _out_call(acc, dinv)
    return out[:N]


# speedup vs baseline: 4.0437x; 1.0327x over previous
import functools

import jax
import jax.numpy as jnp
from jax import lax
from jax.experimental import pallas as pl
from jax.experimental.pallas import tpu as pltpu
from jax.experimental.pallas import tpu_sc as plsc

N_PAD = 10240
D = 128
NC, NS = 2, 16
NW = NC * NS
ROWS_PER_TILE = N_PAD // NS
SLABS = 79
N_REAL = 10000
SLAB_W = 128
LANE = 128
DUMMY = 10200
ROW_BLK = 2000


def _mesh():
    return plsc.VectorSubcoreMesh(core_axis_name="c", subcore_axis_name="s")


def _deg_body(dst_hbm, deg_out, idx_v, ones_v, zeros_v, deg_sh, sem):
    c = lax.axis_index("c")
    s = lax.axis_index("s")
    w = c * NS + s
    for i in range(SLAB_W // 16):
        ones_v[pl.ds(i * 16, 16)] = jnp.full((16,), 1.0, jnp.float32)
    for i in range(ROWS_PER_TILE // 16):
        zeros_v[pl.ds(i * 16, 16)] = jnp.zeros((16,), jnp.float32)
    pltpu.sync_copy(zeros_v, deg_sh.at[pl.ds(s * ROWS_PER_TILE, ROWS_PER_TILE)])
    pltpu.async_copy(dst_hbm.at[w], idx_v, sem).wait()
    plsc.subcore_barrier()

    @pl.loop(0, SLABS)
    def _(j):
        pltpu.sync_copy(ones_v, deg_sh.at[idx_v.at[j]], add=True)

    plsc.subcore_barrier()
    pltpu.sync_copy(deg_sh.at[pl.ds(s * ROWS_PER_TILE, ROWS_PER_TILE)],
                    deg_out.at[c, pl.ds(s * ROWS_PER_TILE, ROWS_PER_TILE)])


@functools.partial(jax.jit)
def _deg_call(dst_slab):
    f = pl.kernel(
        _deg_body,
        out_type=jax.ShapeDtypeStruct((NC, N_PAD), jnp.float32),
        mesh=_mesh(),
        scratch_types=[
            pltpu.VMEM((SLABS, SLAB_W), jnp.int32),
            pltpu.VMEM((SLAB_W,), jnp.float32),
            pltpu.VMEM((ROWS_PER_TILE,), jnp.float32),
            pltpu.VMEM_SHARED((N_PAD,), jnp.float32),
            pltpu.SemaphoreType.DMA,
        ],
    )
    return f(dst_slab)


def _y_body(x_ref, w_ref, deg_ref, y_ref, dinv_ref):
    d = deg_ref[0] + deg_ref[1] + 1.0
    dinv = lax.rsqrt(d)
    xw = jnp.dot(x_ref[...], w_ref[...], preferred_element_type=jnp.float32)
    y_ref[...] = xw * dinv
    dinv_ref[...] = dinv


@jax.jit
def _y_call(x, W, deg3):
    grid = N_REAL // ROW_BLK
    return pl.pallas_call(
        _y_body,
        grid=(grid,),
        in_specs=[
            pl.BlockSpec((ROW_BLK, D), lambda i: (i, 0)),
            pl.BlockSpec((D, D), lambda i: (0, 0)),
            pl.BlockSpec((NC, ROW_BLK, 1), lambda i: (0, i, 0)),
        ],
        out_specs=[
            pl.BlockSpec((ROW_BLK, D), lambda i: (i, 0)),
            pl.BlockSpec((ROW_BLK, 1), lambda i: (i, 0)),
        ],
        out_shape=[
            jax.ShapeDtypeStruct((N_REAL, D), jnp.float32),
            jax.ShapeDtypeStruct((N_REAL, 1), jnp.float32),
        ],
    )(x, W, deg3)


def _msg_body(y_hbm, src_hbm, dst_hbm, acc_out,
              src_v, dst_v, buf0, acc_sh, sem, sem0):
    c = lax.axis_index("c")
    s = lax.axis_index("s")
    w = c * NS + s

    @pl.loop(0, SLAB_W)
    def _(i):
        for k in range(D // 16):
            buf0[i, pl.ds(k * 16, 16)] = jnp.zeros((16,), jnp.float32)

    base = s * ROWS_PER_TILE
    for t in range(ROWS_PER_TILE // SLAB_W):
        pltpu.sync_copy(buf0, acc_sh.at[pl.ds(base + t * SLAB_W, SLAB_W)])

    @pl.when(c == 0)
    def _():
        @pl.when(s < NS - 1)
        def _():
            pltpu.sync_copy(y_hbm.at[pl.ds(base, ROWS_PER_TILE)],
                            acc_sh.at[pl.ds(base, ROWS_PER_TILE)])

        @pl.when(s == NS - 1)
        def _():
            pltpu.sync_copy(y_hbm.at[pl.ds(base, N_REAL - (NS - 1) * ROWS_PER_TILE)],
                            acc_sh.at[pl.ds(base, N_REAL - (NS - 1) * ROWS_PER_TILE)])

    pltpu.async_copy(src_hbm.at[w], src_v, sem).wait()
    pltpu.async_copy(dst_hbm.at[w], dst_v, sem).wait()
    plsc.subcore_barrier()

    @pl.loop(0, SLABS)
    def _(j):
        pltpu.async_copy(y_hbm.at[src_v.at[j]], buf0, sem0).wait()
        pltpu.sync_copy(buf0, acc_sh.at[dst_v.at[j]], add=True)

    plsc.subcore_barrier()
    pltpu.sync_copy(acc_sh.at[pl.ds(s * ROWS_PER_TILE, ROWS_PER_TILE)],
                    acc_out.at[c, pl.ds(s * ROWS_PER_TILE, ROWS_PER_TILE)])


@jax.jit
def _msg_call(y, src_slab, dst_slab):
    f = pl.kernel(
        _msg_body,
        out_type=jax.ShapeDtypeStruct((NC, N_PAD, D), jnp.float32),
        mesh=_mesh(),
        scratch_types=[
            pltpu.VMEM((SLABS, SLAB_W), jnp.int32),
            pltpu.VMEM((SLABS, SLAB_W), jnp.int32),
            pltpu.VMEM((SLAB_W, D), jnp.float32),
            pltpu.VMEM_SHARED((N_PAD, D), jnp.float32),
            pltpu.SemaphoreType.DMA,
            pltpu.SemaphoreType.DMA,
        ],
    )
    return f(y, src_slab, dst_slab)


def _out_body(acc_ref, dinv_ref, o_ref):
    o_ref[...] = jnp.maximum((acc_ref[0] + acc_ref[1]) * dinv_ref[...], 0.0)


@jax.jit
def _out_call(acc, dinv):
    grid = N_REAL // ROW_BLK
    return pl.pallas_call(
        _out_body,
        grid=(grid,),
        in_specs=[
            pl.BlockSpec((NC, ROW_BLK, D), lambda i: (0, i, 0)),
            pl.BlockSpec((ROW_BLK, 1), lambda i: (i, 0)),
        ],
        out_specs=pl.BlockSpec((ROW_BLK, D), lambda i: (i, 0)),
        out_shape=jax.ShapeDtypeStruct((N_REAL, D), jnp.float32),
    )(acc, dinv)


def kernel(x, edge_index, W):
    N = x.shape[0]
    src = edge_index[0].astype(jnp.int32)
    dst = edge_index[1].astype(jnp.int32)
    n_slots = NW * SLABS * SLAB_W
    pad_n = n_slots - src.shape[0]
    pad_ar = jnp.arange(pad_n, dtype=jnp.int32)
    src_all = jnp.concatenate([src, pad_ar % N])
    dst_all = jnp.concatenate([dst, N + pad_ar % (N_PAD - N)])
    src_slab = src_all.reshape(NW, SLABS, SLAB_W)
    dst_slab = dst_all.reshape(NW, SLABS, SLAB_W)

    deg_part = _deg_call(dst_slab)
    deg3 = deg_part.reshape(NC, N_PAD, 1)
    y, dinv = _y_call(x, W, deg3)
    acc = _msg_call(y, src_slab, dst_slab)
    return _out_call(acc, dinv)
